# baseline (speedup 1.0000x reference)
import functools

import jax
import jax.numpy as jnp
from jax import lax
from jax.experimental import pallas as pl
from jax.experimental.pallas import tpu as pltpu
from jax.experimental.pallas import tpu_sc as plsc

NUM_USERS = 100000
NUM_ITEMS = 100000
EMBED_DIM = 64
BATCH = 4096

_INFO = plsc.get_sparse_core_info()
_NC, _NS, _L = _INFO.num_cores, _INFO.num_subcores, _INFO.num_lanes
_NW = _NC * _NS
_BPW = BATCH // _NW
_GROUPS = _BPW // _L


def _sc_kernel(user_hbm, item_hbm, uid_hbm, iid_hbm, out_hbm,
               uid_v, iid_v, gb_v, gc_v, pa_v, pb_v, pc_v, pd_v,
               cu1_v, cu2_v, ci1_v, ci2_v,
               rows_ua, rows_ub, rows_ia, rows_ib, out_v, sem):
    wid = lax.axis_index("s") * _NC + lax.axis_index("c")
    base = wid * _BPW

    pltpu.sync_copy(uid_hbm.at[pl.ds(base, _BPW)], uid_v)
    pltpu.sync_copy(iid_hbm.at[pl.ds(base, _BPW)], iid_v)

    for g in range(_GROUPS):
        sl = pl.ds(g * _L, _L)
        u = uid_v[sl]
        su = jnp.logical_and(jnp.equal(jnp.remainder(u, 1500), 0),
                             u <= 94500)
        gb_v[sl] = jnp.where(su, u + 3, u)
        half = jnp.full((_L,), 0.5, jnp.float32)
        quarter = jnp.full((_L,), 0.25, jnp.float32)
        zero = jnp.zeros((_L,), jnp.float32)
        cu1_v[sl] = jnp.where(su, half, quarter)
        cu2_v[sl] = jnp.where(su, half, zero)

        i = iid_v[sl]
        si = jnp.logical_and(
            jnp.logical_and(jnp.equal(jnp.remainder(i - 3, 1500), 0), i >= 3),
            i <= 94503)
        gc_v[sl] = jnp.where(si, i - 3, i)
        ci1_v[sl] = jnp.where(si, half, zero)
        ci2_v[sl] = jnp.where(si, half, quarter)

    for g in range(_GROUPS):
        sl = pl.ds(g * _L, _L)
        pa_v[sl] = jnp.right_shift(uid_v[sl], 1)
        pb_v[sl] = jnp.right_shift(gb_v[sl], 1)
        pc_v[sl] = jnp.right_shift(gc_v[sl], 1)
        pd_v[sl] = jnp.right_shift(iid_v[sl], 1)

    c1 = pltpu.async_copy(user_hbm.at[pa_v], rows_ua, sem)
    c2 = pltpu.async_copy(item_hbm.at[pb_v], rows_ub, sem)
    c3 = pltpu.async_copy(user_hbm.at[pc_v], rows_ia, sem)
    c4 = pltpu.async_copy(item_hbm.at[pd_v], rows_ib, sem)
    c1.wait(); c2.wait(); c3.wait(); c4.wait()

    lane = lax.iota(jnp.int32, _L)
    for g in range(_GROUPS):
        sl = pl.ds(g * _L, _L)
        lrow = lane + g * _L
        cu1 = cu1_v[sl]
        cu2 = cu2_v[sl]
        ci1 = ci1_v[sl]
        ci2 = ci2_v[sl]
        ha = jnp.left_shift(jnp.bitwise_and(uid_v[sl], 1), 6)
        hb = jnp.left_shift(jnp.bitwise_and(gb_v[sl], 1), 6)
        hc = jnp.left_shift(jnp.bitwise_and(gc_v[sl], 1), 6)
        hd = jnp.left_shift(jnp.bitwise_and(iid_v[sl], 1), 6)

        def body(d, acc):
            col = jnp.bitwise_and(lane + d, EMBED_DIM - 1)
            ua = plsc.load_gather(rows_ua, [lrow, col + ha])
            ub = plsc.load_gather(rows_ub, [lrow, col + hb])
            ia = plsc.load_gather(rows_ia, [lrow, col + hc])
            ib = plsc.load_gather(rows_ib, [lrow, col + hd])
            ue = cu1 * ua + cu2 * ub
            ie = ci1 * ia + ci2 * ib
            return acc + ue * ie

        out_v[sl] = lax.fori_loop(0, EMBED_DIM, body,
                                  jnp.zeros((_L,), jnp.float32))

    pltpu.sync_copy(out_v, out_hbm.at[pl.ds(base, _BPW)])


@jax.jit
def _run(user_emb, item_emb, user_ids, item_ids):
    mesh = plsc.VectorSubcoreMesh(core_axis_name="c", subcore_axis_name="s")
    kern = functools.partial(
        pl.kernel,
        mesh=mesh,
        compiler_params=pltpu.CompilerParams(
            needs_layout_passes=False, use_tc_tiling_on_sc=True),
        out_type=jax.ShapeDtypeStruct((BATCH,), jnp.float32),
        scratch_types=[
            pltpu.VMEM((_BPW,), jnp.int32),
            pltpu.VMEM((_BPW,), jnp.int32),
            pltpu.VMEM((_BPW,), jnp.int32),
            pltpu.VMEM((_BPW,), jnp.int32),
            pltpu.VMEM((_BPW,), jnp.int32),
            pltpu.VMEM((_BPW,), jnp.int32),
            pltpu.VMEM((_BPW,), jnp.int32),
            pltpu.VMEM((_BPW,), jnp.int32),
            pltpu.VMEM((_BPW,), jnp.float32),
            pltpu.VMEM((_BPW,), jnp.float32),
            pltpu.VMEM((_BPW,), jnp.float32),
            pltpu.VMEM((_BPW,), jnp.float32),
            pltpu.VMEM((_BPW, 2 * EMBED_DIM), jnp.float32),
            pltpu.VMEM((_BPW, 2 * EMBED_DIM), jnp.float32),
            pltpu.VMEM((_BPW, 2 * EMBED_DIM), jnp.float32),
            pltpu.VMEM((_BPW, 2 * EMBED_DIM), jnp.float32),
            pltpu.VMEM((_BPW,), jnp.float32),
            pltpu.SemaphoreType.DMA,
        ],
    )(_sc_kernel)
    u2 = user_emb.reshape(NUM_USERS // 2, 2 * EMBED_DIM)
    i2 = item_emb.reshape(NUM_ITEMS // 2, 2 * EMBED_DIM)
    return kern(u2, i2, user_ids, item_ids)


def kernel(user_emb, item_emb, user_ids, item_ids):
    return _run(user_emb, item_emb,
                user_ids.astype(jnp.int32), item_ids.astype(jnp.int32))


# speedup vs baseline: 1.3718x; 1.3718x over previous
import functools

import jax
import jax.numpy as jnp
from jax import lax
from jax.experimental import pallas as pl
from jax.experimental.pallas import tpu as pltpu
from jax.experimental.pallas import tpu_sc as plsc

NUM_USERS = 100000
NUM_ITEMS = 100000
EMBED_DIM = 64
BATCH = 4096

_INFO = plsc.get_sparse_core_info()
_NC, _NS, _L = _INFO.num_cores, _INFO.num_subcores, _INFO.num_lanes
_NW = _NC * _NS
_BPW = BATCH // _NW
_GROUPS = _BPW // _L


def _sc_kernel(user_hbm, item_hbm, uid_hbm, iid_hbm, out_hbm,
               uid_v, iid_v, gb_v, gc_v,
               cu1_v, cu2_v, ci1_v, ci2_v,
               rows_ua, rows_ub, rows_ia, rows_ib, out_v, sem):
    wid = lax.axis_index("s") * _NC + lax.axis_index("c")
    base = wid * _BPW

    pltpu.sync_copy(uid_hbm.at[pl.ds(base, _BPW)], uid_v)
    pltpu.sync_copy(iid_hbm.at[pl.ds(base, _BPW)], iid_v)

    for g in range(_GROUPS):
        sl = pl.ds(g * _L, _L)
        u = uid_v[sl]
        su = jnp.logical_and(jnp.equal(jnp.remainder(u, 1500), 0),
                             u <= 94500)
        gb_v[sl] = jnp.where(su, u + 3, jnp.full((_L,), -1, jnp.int32))
        half = jnp.full((_L,), 0.5, jnp.float32)
        quarter = jnp.full((_L,), 0.25, jnp.float32)
        zero = jnp.zeros((_L,), jnp.float32)
        cu1_v[sl] = jnp.where(su, half, quarter)
        cu2_v[sl] = jnp.where(su, half, zero)

        i = iid_v[sl]
        si = jnp.logical_and(
            jnp.logical_and(jnp.equal(jnp.remainder(i - 3, 1500), 0), i >= 3),
            i <= 94503)
        gc_v[sl] = jnp.where(si, i - 3, jnp.full((_L,), -1, jnp.int32))
        ci1_v[sl] = jnp.where(si, half, zero)
        ci2_v[sl] = jnp.where(si, half, quarter)

    zrow = jnp.zeros((_L,), jnp.float32)

    def zinit(b, _):
        for c in range(EMBED_DIM // _L):
            rows_ub[b, pl.ds(c * _L, _L)] = zrow
            rows_ia[b, pl.ds(c * _L, _L)] = zrow
        return ()

    lax.fori_loop(0, _BPW, zinit, ())

    nspec = jnp.int32(0)
    for g in range(_GROUPS):
        sl = pl.ds(g * _L, _L)
        ga_c = uid_v[sl]
        gb_c = gb_v[sl]
        gc_c = gc_v[sl]
        gd_c = iid_v[sl]
        for j in range(_L):
            b = g * _L + j
            pltpu.async_copy(user_hbm.at[pl.ds(ga_c[j], 1)],
                             rows_ua.at[pl.ds(b, 1)], sem)
            pltpu.async_copy(item_hbm.at[pl.ds(gd_c[j], 1)],
                             rows_ib.at[pl.ds(b, 1)], sem)
            gbj = gb_c[j]
            gcj = gc_c[j]

            @pl.when(gbj >= 0)
            def _():
                pltpu.async_copy(item_hbm.at[pl.ds(gbj, 1)],
                                 rows_ub.at[pl.ds(b, 1)], sem)

            @pl.when(gcj >= 0)
            def _():
                pltpu.async_copy(user_hbm.at[pl.ds(gcj, 1)],
                                 rows_ia.at[pl.ds(b, 1)], sem)

            nspec = nspec + jnp.where(gbj >= 0, 1, 0) \
                          + jnp.where(gcj >= 0, 1, 0)

    pltpu.make_async_copy(user_hbm.at[pl.ds(0, _BPW)], rows_ua, sem).wait()
    pltpu.make_async_copy(item_hbm.at[pl.ds(0, _BPW)], rows_ib, sem).wait()

    def drain(_, __):
        pltpu.make_async_copy(user_hbm.at[pl.ds(0, 1)],
                              rows_ub.at[pl.ds(0, 1)], sem).wait()
        return ()

    lax.fori_loop(0, nspec, drain, ())

    lane = lax.iota(jnp.int32, _L)
    for g in range(_GROUPS):
        sl = pl.ds(g * _L, _L)
        lrow = lane + g * _L
        cu1 = cu1_v[sl]
        cu2 = cu2_v[sl]
        ci1 = ci1_v[sl]
        ci2 = ci2_v[sl]

        def body(d, acc):
            col = jnp.bitwise_and(lane + d, EMBED_DIM - 1)
            ua = plsc.load_gather(rows_ua, [lrow, col])
            ub = plsc.load_gather(rows_ub, [lrow, col])
            ia = plsc.load_gather(rows_ia, [lrow, col])
            ib = plsc.load_gather(rows_ib, [lrow, col])
            ue = cu1 * ua + cu2 * ub
            ie = ci1 * ia + ci2 * ib
            return acc + ue * ie

        out_v[sl] = lax.fori_loop(0, EMBED_DIM, body,
                                  jnp.zeros((_L,), jnp.float32))

    pltpu.sync_copy(out_v, out_hbm.at[pl.ds(base, _BPW)])


@jax.jit
def _run(user_emb, item_emb, user_ids, item_ids):
    mesh = plsc.VectorSubcoreMesh(core_axis_name="c", subcore_axis_name="s")
    kern = functools.partial(
        pl.kernel,
        mesh=mesh,
        compiler_params=pltpu.CompilerParams(
            needs_layout_passes=False, use_tc_tiling_on_sc=True),
        out_type=jax.ShapeDtypeStruct((BATCH,), jnp.float32),
        scratch_types=[
            pltpu.VMEM((_BPW,), jnp.int32),
            pltpu.VMEM((_BPW,), jnp.int32),
            pltpu.VMEM((_BPW,), jnp.int32),
            pltpu.VMEM((_BPW,), jnp.int32),
            pltpu.VMEM((_BPW,), jnp.float32),
            pltpu.VMEM((_BPW,), jnp.float32),
            pltpu.VMEM((_BPW,), jnp.float32),
            pltpu.VMEM((_BPW,), jnp.float32),
            pltpu.VMEM((_BPW, EMBED_DIM), jnp.float32),
            pltpu.VMEM((_BPW, EMBED_DIM), jnp.float32),
            pltpu.VMEM((_BPW, EMBED_DIM), jnp.float32),
            pltpu.VMEM((_BPW, EMBED_DIM), jnp.float32),
            pltpu.VMEM((_BPW,), jnp.float32),
            pltpu.SemaphoreType.DMA,
        ],
    )(_sc_kernel)
    return kern(user_emb, item_emb, user_ids, item_ids)


def kernel(user_emb, item_emb, user_ids, item_ids):
    return _run(user_emb, item_emb,
                user_ids.astype(jnp.int32), item_ids.astype(jnp.int32))


# speedup vs baseline: 1.4123x; 1.0295x over previous
import functools

import jax
import jax.numpy as jnp
from jax import lax
from jax.experimental import pallas as pl
from jax.experimental.pallas import tpu as pltpu
from jax.experimental.pallas import tpu_sc as plsc

NUM_USERS = 100000
NUM_ITEMS = 100000
EMBED_DIM = 64
BATCH = 4096

_INFO = plsc.get_sparse_core_info()
_NC, _NS, _L = _INFO.num_cores, _INFO.num_subcores, _INFO.num_lanes
_NW = _NC * _NS
_BPW = BATCH // _NW
_GROUPS = _BPW // _L


def _sc_kernel(user_hbm, item_hbm, uid_hbm, iid_hbm, out_hbm,
               uid_v, iid_v, gb_v, gc_v,
               cu1_v, cu2_v, ci1_v, ci2_v,
               rows_ua, rows_ub, rows_ia, rows_ib, out_v, sem):
    wid = lax.axis_index("s") * _NC + lax.axis_index("c")
    base = wid * _BPW

    pltpu.sync_copy(uid_hbm.at[pl.ds(base, _BPW)], uid_v)
    pltpu.sync_copy(iid_hbm.at[pl.ds(base, _BPW)], iid_v)

    for g in range(_GROUPS):
        sl = pl.ds(g * _L, _L)
        u = uid_v[sl]
        su = jnp.logical_and(jnp.equal(jnp.remainder(u, 1500), 0),
                             u <= 94500)
        gb_v[sl] = jnp.where(su, u + 3, u)
        half = jnp.full((_L,), 0.5, jnp.float32)
        quarter = jnp.full((_L,), 0.25, jnp.float32)
        zero = jnp.zeros((_L,), jnp.float32)
        cu1_v[sl] = jnp.where(su, half, quarter)
        cu2_v[sl] = jnp.where(su, half, zero)

        i = iid_v[sl]
        si = jnp.logical_and(
            jnp.logical_and(jnp.equal(jnp.remainder(i - 3, 1500), 0), i >= 3),
            i <= 94503)
        gc_v[sl] = jnp.where(si, i - 3, i)
        ci1_v[sl] = jnp.where(si, half, zero)
        ci2_v[sl] = jnp.where(si, half, quarter)

    for g in range(_GROUPS):
        sl = pl.ds(g * _L, _L)
        ga_c = uid_v[sl]
        gb_c = gb_v[sl]
        gc_c = gc_v[sl]
        gd_c = iid_v[sl]
        for j in range(_L):
            b = g * _L + j
            pltpu.async_copy(user_hbm.at[pl.ds(ga_c[j], 1)],
                             rows_ua.at[pl.ds(b, 1)], sem)
            pltpu.async_copy(item_hbm.at[pl.ds(gb_c[j], 1)],
                             rows_ub.at[pl.ds(b, 1)], sem)
            pltpu.async_copy(user_hbm.at[pl.ds(gc_c[j], 1)],
                             rows_ia.at[pl.ds(b, 1)], sem)
            pltpu.async_copy(item_hbm.at[pl.ds(gd_c[j], 1)],
                             rows_ib.at[pl.ds(b, 1)], sem)

    for buf in (rows_ua, rows_ub, rows_ia, rows_ib):
        pltpu.make_async_copy(user_hbm.at[pl.ds(0, _BPW)], buf, sem).wait()

    lane = lax.iota(jnp.int32, _L)
    for g in range(_GROUPS):
        sl = pl.ds(g * _L, _L)
        lrow = lane + g * _L
        cu1 = cu1_v[sl]
        cu2 = cu2_v[sl]
        ci1 = ci1_v[sl]
        ci2 = ci2_v[sl]

        def body(d, acc):
            col = jnp.bitwise_and(lane + d, EMBED_DIM - 1)
            ua = plsc.load_gather(rows_ua, [lrow, col])
            ub = plsc.load_gather(rows_ub, [lrow, col])
            ia = plsc.load_gather(rows_ia, [lrow, col])
            ib = plsc.load_gather(rows_ib, [lrow, col])
            ue = cu1 * ua + cu2 * ub
            ie = ci1 * ia + ci2 * ib
            return acc + ue * ie

        out_v[sl] = lax.fori_loop(0, EMBED_DIM, body,
                                  jnp.zeros((_L,), jnp.float32))

    pltpu.sync_copy(out_v, out_hbm.at[pl.ds(base, _BPW)])


@jax.jit
def _run(user_emb, item_emb, user_ids, item_ids):
    mesh = plsc.VectorSubcoreMesh(core_axis_name="c", subcore_axis_name="s")
    kern = functools.partial(
        pl.kernel,
        mesh=mesh,
        compiler_params=pltpu.CompilerParams(
            needs_layout_passes=False, use_tc_tiling_on_sc=True),
        out_type=jax.ShapeDtypeStruct((BATCH,), jnp.float32),
        scratch_types=[
            pltpu.VMEM((_BPW,), jnp.int32),
            pltpu.VMEM((_BPW,), jnp.int32),
            pltpu.VMEM((_BPW,), jnp.int32),
            pltpu.VMEM((_BPW,), jnp.int32),
            pltpu.VMEM((_BPW,), jnp.float32),
            pltpu.VMEM((_BPW,), jnp.float32),
            pltpu.VMEM((_BPW,), jnp.float32),
            pltpu.VMEM((_BPW,), jnp.float32),
            pltpu.VMEM((_BPW, EMBED_DIM), jnp.float32),
            pltpu.VMEM((_BPW, EMBED_DIM), jnp.float32),
            pltpu.VMEM((_BPW, EMBED_DIM), jnp.float32),
            pltpu.VMEM((_BPW, EMBED_DIM), jnp.float32),
            pltpu.VMEM((_BPW,), jnp.float32),
            pltpu.SemaphoreType.DMA,
        ],
    )(_sc_kernel)
    return kern(user_emb, item_emb, user_ids, item_ids)


def kernel(user_emb, item_emb, user_ids, item_ids):
    return _run(user_emb, item_emb,
                user_ids.astype(jnp.int32), item_ids.astype(jnp.int32))


# speedup vs baseline: 1.4524x; 1.0283x over previous
import functools

import jax
import jax.numpy as jnp
from jax import lax
from jax.experimental import pallas as pl
from jax.experimental.pallas import tpu as pltpu
from jax.experimental.pallas import tpu_sc as plsc

NUM_USERS = 100000
NUM_ITEMS = 100000
EMBED_DIM = 64
BATCH = 4096

_INFO = plsc.get_sparse_core_info()
_NC, _NS, _L = _INFO.num_cores, _INFO.num_subcores, _INFO.num_lanes
_NW = _NC * _NS
_BPW = BATCH // _NW
_GROUPS = _BPW // _L


def _sc_kernel(user_hbm, item_hbm, uid_hbm, iid_hbm, out_hbm,
               uid_v, iid_v, gb_v, gc_v,
               cu1_v, cu2_v, ci1_v, ci2_v,
               rows_ua, rows_ub, rows_ia, rows_ib, out_v, sem):
    wid = lax.axis_index("s") * _NC + lax.axis_index("c")
    base = wid * _BPW

    pltpu.sync_copy(uid_hbm.at[pl.ds(base, _BPW)], uid_v.at[pl.ds(0, _BPW)])
    pltpu.sync_copy(iid_hbm.at[pl.ds(base, _BPW)], iid_v.at[pl.ds(0, _BPW)])

    for g in range(_GROUPS):
        sl = pl.ds(g * _L, _L)
        u = uid_v[sl]
        su = jnp.logical_and(jnp.equal(jnp.remainder(u, 1500), 0),
                             u <= 94500)
        gb_v[sl] = jnp.where(su, u + 3, u)
        half = jnp.full((_L,), 0.5, jnp.float32)
        quarter = jnp.full((_L,), 0.25, jnp.float32)
        zero = jnp.zeros((_L,), jnp.float32)
        cu1_v[sl] = jnp.where(su, half, quarter)
        cu2_v[sl] = jnp.where(su, half, zero)

        i = iid_v[sl]
        si = jnp.logical_and(
            jnp.logical_and(jnp.equal(jnp.remainder(i - 3, 1500), 0), i >= 3),
            i <= 94503)
        gc_v[sl] = jnp.where(si, i - 3, i)
        ci1_v[sl] = jnp.where(si, half, zero)
        ci2_v[sl] = jnp.where(si, half, quarter)

    def fire(b, _):
        ga = uid_v[pl.ds(b, _L)][0]
        gb = gb_v[pl.ds(b, _L)][0]
        gc = gc_v[pl.ds(b, _L)][0]
        gd = iid_v[pl.ds(b, _L)][0]
        pltpu.async_copy(user_hbm.at[pl.ds(ga, 1)],
                         rows_ua.at[pl.ds(b, 1)], sem)
        pltpu.async_copy(item_hbm.at[pl.ds(gb, 1)],
                         rows_ub.at[pl.ds(b, 1)], sem)
        pltpu.async_copy(user_hbm.at[pl.ds(gc, 1)],
                         rows_ia.at[pl.ds(b, 1)], sem)
        pltpu.async_copy(item_hbm.at[pl.ds(gd, 1)],
                         rows_ib.at[pl.ds(b, 1)], sem)
        return ()

    lax.fori_loop(0, _BPW, fire, ())
    for buf in (rows_ua, rows_ub, rows_ia, rows_ib):
        pltpu.make_async_copy(user_hbm.at[pl.ds(0, _BPW)], buf, sem).wait()

    lane = lax.iota(jnp.int32, _L)
    for g in range(_GROUPS):
        sl = pl.ds(g * _L, _L)
        lrow = lane + g * _L
        cu1 = cu1_v[sl]
        cu2 = cu2_v[sl]
        ci1 = ci1_v[sl]
        ci2 = ci2_v[sl]

        def body(d, acc):
            col = jnp.bitwise_and(lane + d, EMBED_DIM - 1)
            ua = plsc.load_gather(rows_ua, [lrow, col])
            ub = plsc.load_gather(rows_ub, [lrow, col])
            ia = plsc.load_gather(rows_ia, [lrow, col])
            ib = plsc.load_gather(rows_ib, [lrow, col])
            ue = cu1 * ua + cu2 * ub
            ie = ci1 * ia + ci2 * ib
            return acc + ue * ie

        out_v[sl] = lax.fori_loop(0, EMBED_DIM, body,
                                  jnp.zeros((_L,), jnp.float32))

    pltpu.sync_copy(out_v, out_hbm.at[pl.ds(base, _BPW)])


@jax.jit
def _run(user_emb, item_emb, user_ids, item_ids):
    mesh = plsc.VectorSubcoreMesh(core_axis_name="c", subcore_axis_name="s")
    kern = functools.partial(
        pl.kernel,
        mesh=mesh,
        compiler_params=pltpu.CompilerParams(
            needs_layout_passes=False, use_tc_tiling_on_sc=True),
        out_type=jax.ShapeDtypeStruct((BATCH,), jnp.float32),
        scratch_types=[
            pltpu.VMEM((_BPW + _L,), jnp.int32),
            pltpu.VMEM((_BPW + _L,), jnp.int32),
            pltpu.VMEM((_BPW + _L,), jnp.int32),
            pltpu.VMEM((_BPW + _L,), jnp.int32),
            pltpu.VMEM((_BPW,), jnp.float32),
            pltpu.VMEM((_BPW,), jnp.float32),
            pltpu.VMEM((_BPW,), jnp.float32),
            pltpu.VMEM((_BPW,), jnp.float32),
            pltpu.VMEM((_BPW, EMBED_DIM), jnp.float32),
            pltpu.VMEM((_BPW, EMBED_DIM), jnp.float32),
            pltpu.VMEM((_BPW, EMBED_DIM), jnp.float32),
            pltpu.VMEM((_BPW, EMBED_DIM), jnp.float32),
            pltpu.VMEM((_BPW,), jnp.float32),
            pltpu.SemaphoreType.DMA,
        ],
    )(_sc_kernel)
    return kern(user_emb, item_emb, user_ids, item_ids)


def kernel(user_emb, item_emb, user_ids, item_ids):
    return _run(user_emb, item_emb,
                user_ids.astype(jnp.int32), item_ids.astype(jnp.int32))
